# baseline (speedup 1.0000x reference)
import functools
import math

import jax
import jax.numpy as jnp
from jax import lax
from jax.experimental import pallas as pl
from jax.experimental.pallas import tpu as pltpu
from jax.experimental.pallas import tpu_sc as plsc

D_MODEL = 64
SCALE = math.sqrt(D_MODEL)

NUM_CORES = 2
NUM_SUBCORES = 16
NW = NUM_CORES * NUM_SUBCORES
CHUNK = 256
NBUF = 4
ROW_PAD = 128

TC_BLOCK = 2048


def _pad_scale_body(wt_ref, out_ref):
    d_idx = jax.lax.broadcasted_iota(jnp.int32, (D_MODEL, D_MODEL), 0)
    c_idx = jax.lax.broadcasted_iota(jnp.int32, (D_MODEL, D_MODEL), 1)
    sel = jnp.where(d_idx == c_idx, jnp.float32(SCALE), jnp.float32(0.0))
    t = jax.lax.dot_general(
        wt_ref[...], sel, (((0,), (0,)), ((), ())),
        preferred_element_type=jnp.float32)
    t3 = t.reshape(TC_BLOCK // 2, 2, D_MODEL)
    out_ref[...] = jnp.concatenate([t3[:, 0, :], t3[:, 1, :]], axis=1)


@functools.cache
def _build_pad_scale(V: int):
    grid = (V + TC_BLOCK - 1) // TC_BLOCK
    return pl.pallas_call(
        _pad_scale_body,
        grid=(grid,),
        in_specs=[pl.BlockSpec((D_MODEL, TC_BLOCK), lambda i: (0, i))],
        out_specs=pl.BlockSpec((TC_BLOCK // 2, 2 * D_MODEL), lambda i: (i, 0)),
        out_shape=jax.ShapeDtypeStruct((V // 2, 2 * D_MODEL), jnp.float32),
    )


@functools.cache
def _build_gather(B: int, V: int):
    assert B % (NW * CHUNK) == 0
    b_per_w = B // NW
    n_chunks = b_per_w // CHUNK
    assert n_chunks % NBUF == 0 and n_chunks >= 2 * NBUF
    mesh = plsc.VectorSubcoreMesh(
        core_axis_name="c", subcore_axis_name="s",
        num_cores=NUM_CORES, num_subcores=NUM_SUBCORES)

    @functools.partial(
        pl.kernel,
        out_type=jax.ShapeDtypeStruct((B, ROW_PAD), jnp.float32),
        mesh=mesh,
        scratch_types=[
            pltpu.VMEM((b_per_w,), jnp.int32),
            [pltpu.VMEM((CHUNK, D_MODEL), jnp.float32)] * NBUF,
            [pltpu.SemaphoreType.DMA] * NBUF,
            [pltpu.SemaphoreType.DMA] * NBUF,
        ],
        compiler_params=pltpu.CompilerParams(use_tc_tiling_on_sc=False),
    )
    def emb(x_hbm, w_hbm, out_hbm, idx_v, rows, gsem, ssem):
        wid = lax.axis_index("s") * NUM_CORES + lax.axis_index("c")
        base = wid * b_per_w

        def gather(g, b):
            pltpu.async_copy(
                w_hbm.at[idx_v.at[pl.ds(g * CHUNK, CHUNK)]], rows[b], gsem[b])

        def wait_gather(b):
            pltpu.make_async_copy(
                w_hbm.at[idx_v.at[pl.ds(0, CHUNK)]], rows[b], gsem[b]).wait()

        def scatter(g, b):
            pltpu.async_copy(
                rows[b],
                out_hbm.at[pl.ds(base + g * CHUNK, CHUNK), pl.ds(0, D_MODEL)],
                ssem[b])

        def wait_scatter(b):
            pltpu.make_async_copy(
                rows[b],
                out_hbm.at[pl.ds(base, CHUNK), pl.ds(0, D_MODEL)],
                ssem[b]).wait()

        pltpu.sync_copy(x_hbm.at[pl.ds(base, b_per_w)], idx_v)

        for b in range(2):
            gather(b, b)

        @pl.loop(0, n_chunks, step=NBUF)
        def _chunks(g0):
            for b in range(NBUF):
                g = g0 + b
                pf = (b + 2) % NBUF
                @pl.when(g + 2 < n_chunks)
                def _():
                    @pl.when(g >= 2)
                    def _():
                        wait_scatter(pf)
                    gather(g + 2, pf)

                wait_gather(b)
                scatter(g, b)

        for b in range(NBUF):
            wait_scatter(b)

    return emb


def kernel(x, weight):
    batch, seq = x.shape
    vocab, _ = weight.shape
    wt = jnp.swapaxes(weight, 0, 1)
    staged = _build_pad_scale(vocab)(wt).reshape(vocab, D_MODEL)
    flat = x.reshape(-1).astype(jnp.int32)
    out = _build_gather(batch * seq, vocab)(flat, staged)
    return out[:, :D_MODEL].reshape(batch, seq, D_MODEL)


# speedup vs baseline: 1.1980x; 1.1980x over previous
import functools
import math

import jax
import jax.numpy as jnp
from jax import lax
from jax.experimental import pallas as pl
from jax.experimental.pallas import tpu as pltpu
from jax.experimental.pallas import tpu_sc as plsc

D_MODEL = 64
SCALE = math.sqrt(D_MODEL)

NUM_CORES = 2
NUM_SUBCORES = 16
NW = NUM_CORES * NUM_SUBCORES
CHUNK = 256
NBUF = 4
ROW_PAD = 128

TC_BLOCK = 4096


def _pad_scale_body(wt_ref, out_ref):
    t = (wt_ref[...] * jnp.float32(SCALE)).T
    t3 = t.reshape(TC_BLOCK // 2, 2, D_MODEL)
    out_ref[...] = jnp.concatenate([t3[:, 0, :], t3[:, 1, :]], axis=1)


@functools.cache
def _build_pad_scale(V: int):
    grid = (V + TC_BLOCK - 1) // TC_BLOCK
    return pl.pallas_call(
        _pad_scale_body,
        grid=(grid,),
        in_specs=[pl.BlockSpec((D_MODEL, TC_BLOCK), lambda i: (0, i))],
        out_specs=pl.BlockSpec((TC_BLOCK // 2, 2 * D_MODEL), lambda i: (i, 0)),
        out_shape=jax.ShapeDtypeStruct((V // 2, 2 * D_MODEL), jnp.float32),
    )


@functools.cache
def _build_gather(B: int, V: int):
    assert B % (NW * CHUNK) == 0
    b_per_w = B // NW
    n_chunks = b_per_w // CHUNK
    assert n_chunks % NBUF == 0 and n_chunks >= 2 * NBUF
    mesh = plsc.VectorSubcoreMesh(
        core_axis_name="c", subcore_axis_name="s",
        num_cores=NUM_CORES, num_subcores=NUM_SUBCORES)

    @functools.partial(
        pl.kernel,
        out_type=jax.ShapeDtypeStruct((B, ROW_PAD), jnp.float32),
        mesh=mesh,
        scratch_types=[
            pltpu.VMEM((b_per_w,), jnp.int32),
            [pltpu.VMEM((CHUNK, D_MODEL), jnp.float32)] * NBUF,
            [pltpu.SemaphoreType.DMA] * NBUF,
            [pltpu.SemaphoreType.DMA] * NBUF,
        ],
        compiler_params=pltpu.CompilerParams(use_tc_tiling_on_sc=False),
    )
    def emb(x_hbm, w_hbm, out_hbm, idx_v, rows, gsem, ssem):
        wid = lax.axis_index("s") * NUM_CORES + lax.axis_index("c")
        base = wid * b_per_w

        def gather(g, b):
            pltpu.async_copy(
                w_hbm.at[idx_v.at[pl.ds(g * CHUNK, CHUNK)]], rows[b], gsem[b])

        def wait_gather(b):
            pltpu.make_async_copy(
                w_hbm.at[idx_v.at[pl.ds(0, CHUNK)]], rows[b], gsem[b]).wait()

        def scatter(g, b):
            pltpu.async_copy(
                rows[b],
                out_hbm.at[pl.ds(base + g * CHUNK, CHUNK), pl.ds(0, D_MODEL)],
                ssem[b])

        def wait_scatter(b):
            pltpu.make_async_copy(
                rows[b],
                out_hbm.at[pl.ds(base, CHUNK), pl.ds(0, D_MODEL)],
                ssem[b]).wait()

        pltpu.sync_copy(x_hbm.at[pl.ds(base, b_per_w)], idx_v)

        for b in range(2):
            gather(b, b)

        @pl.loop(0, n_chunks, step=NBUF)
        def _chunks(g0):
            for b in range(NBUF):
                g = g0 + b
                pf = (b + 2) % NBUF
                @pl.when(g + 2 < n_chunks)
                def _():
                    @pl.when(g >= 2)
                    def _():
                        wait_scatter(pf)
                    gather(g + 2, pf)

                wait_gather(b)
                scatter(g, b)

        for b in range(NBUF):
            wait_scatter(b)

    return emb


def kernel(x, weight):
    batch, seq = x.shape
    vocab, _ = weight.shape
    wt = jnp.swapaxes(weight, 0, 1)
    staged = _build_pad_scale(vocab)(wt).reshape(vocab, D_MODEL)
    flat = x.reshape(-1).astype(jnp.int32)
    out = _build_gather(batch * seq, vocab)(flat, staged)
    return out[:, :D_MODEL].reshape(batch, seq, D_MODEL)


# speedup vs baseline: 1.2326x; 1.0288x over previous
import functools
import math

import jax
import jax.numpy as jnp
from jax import lax
from jax.experimental import pallas as pl
from jax.experimental.pallas import tpu as pltpu
from jax.experimental.pallas import tpu_sc as plsc

D_MODEL = 64
SCALE = math.sqrt(D_MODEL)

NUM_CORES = 2
NUM_SUBCORES = 16
NW = NUM_CORES * NUM_SUBCORES
CHUNK = 256
NBUF = 4
ROW_PAD = 128

TC_BLOCK = 8192


def _pad_scale_body(wt_ref, out_ref):
    t = (wt_ref[...] * jnp.float32(SCALE)).T
    t3 = t.reshape(TC_BLOCK // 2, 2, D_MODEL)
    out_ref[...] = jnp.concatenate([t3[:, 0, :], t3[:, 1, :]], axis=1)


@functools.cache
def _build_pad_scale(V: int):
    grid = (V + TC_BLOCK - 1) // TC_BLOCK
    return pl.pallas_call(
        _pad_scale_body,
        grid=(grid,),
        in_specs=[pl.BlockSpec((D_MODEL, TC_BLOCK), lambda i: (0, i))],
        out_specs=pl.BlockSpec((TC_BLOCK // 2, 2 * D_MODEL), lambda i: (i, 0)),
        out_shape=jax.ShapeDtypeStruct((V // 2, 2 * D_MODEL), jnp.float32),
    )


@functools.cache
def _build_gather(B: int, V: int):
    assert B % (NW * CHUNK) == 0
    b_per_w = B // NW
    n_chunks = b_per_w // CHUNK
    assert n_chunks % NBUF == 0 and n_chunks >= 2 * NBUF
    mesh = plsc.VectorSubcoreMesh(
        core_axis_name="c", subcore_axis_name="s",
        num_cores=NUM_CORES, num_subcores=NUM_SUBCORES)

    @functools.partial(
        pl.kernel,
        out_type=jax.ShapeDtypeStruct((B, ROW_PAD), jnp.float32),
        mesh=mesh,
        scratch_types=[
            pltpu.VMEM((b_per_w,), jnp.int32),
            [pltpu.VMEM((CHUNK, D_MODEL), jnp.float32)] * NBUF,
            [pltpu.SemaphoreType.DMA] * NBUF,
            [pltpu.SemaphoreType.DMA] * NBUF,
        ],
        compiler_params=pltpu.CompilerParams(use_tc_tiling_on_sc=False),
    )
    def emb(x_hbm, w_hbm, out_hbm, idx_v, rows, gsem, ssem):
        wid = lax.axis_index("s") * NUM_CORES + lax.axis_index("c")
        base = wid * b_per_w

        def gather(g, b):
            pltpu.async_copy(
                w_hbm.at[idx_v.at[pl.ds(g * CHUNK, CHUNK)]], rows[b], gsem[b])

        def wait_gather(b):
            pltpu.make_async_copy(
                w_hbm.at[idx_v.at[pl.ds(0, CHUNK)]], rows[b], gsem[b]).wait()

        def scatter(g, b):
            pltpu.async_copy(
                rows[b],
                out_hbm.at[pl.ds(base + g * CHUNK, CHUNK), pl.ds(0, D_MODEL)],
                ssem[b])

        def wait_scatter(b):
            pltpu.make_async_copy(
                rows[b],
                out_hbm.at[pl.ds(base, CHUNK), pl.ds(0, D_MODEL)],
                ssem[b]).wait()

        pltpu.sync_copy(x_hbm.at[pl.ds(base, b_per_w)], idx_v)

        for b in range(2):
            gather(b, b)

        @pl.loop(0, n_chunks, step=NBUF)
        def _chunks(g0):
            for b in range(NBUF):
                g = g0 + b
                pf = (b + 2) % NBUF
                @pl.when(g + 2 < n_chunks)
                def _():
                    @pl.when(g >= 2)
                    def _():
                        wait_scatter(pf)
                    gather(g + 2, pf)

                wait_gather(b)
                scatter(g, b)

        for b in range(NBUF):
            wait_scatter(b)

    return emb


def kernel(x, weight):
    batch, seq = x.shape
    vocab, _ = weight.shape
    wt = jnp.swapaxes(weight, 0, 1)
    staged = _build_pad_scale(vocab)(wt).reshape(vocab, D_MODEL)
    flat = x.reshape(-1).astype(jnp.int32)
    out = _build_gather(batch * seq, vocab)(flat, staged)
    return out[:, :D_MODEL].reshape(batch, seq, D_MODEL)


# speedup vs baseline: 1.2351x; 1.0020x over previous
import functools
import math

import jax
import jax.numpy as jnp
from jax import lax
from jax.experimental import pallas as pl
from jax.experimental.pallas import tpu as pltpu
from jax.experimental.pallas import tpu_sc as plsc

D_MODEL = 64
SCALE = math.sqrt(D_MODEL)

NUM_CORES = 2
NUM_SUBCORES = 16
NW = NUM_CORES * NUM_SUBCORES
CHUNK = 256
NBUF = 4
ROW_PAD = 128

TC_BLOCK = 16384


def _pad_scale_body(wt_ref, out_ref):
    t = (wt_ref[...] * jnp.float32(SCALE)).T
    t3 = t.reshape(TC_BLOCK // 2, 2, D_MODEL)
    out_ref[...] = jnp.concatenate([t3[:, 0, :], t3[:, 1, :]], axis=1)


@functools.cache
def _build_pad_scale(V: int):
    grid = (V + TC_BLOCK - 1) // TC_BLOCK
    return pl.pallas_call(
        _pad_scale_body,
        grid=(grid,),
        in_specs=[pl.BlockSpec((D_MODEL, TC_BLOCK), lambda i: (0, i))],
        out_specs=pl.BlockSpec((TC_BLOCK // 2, 2 * D_MODEL), lambda i: (i, 0)),
        out_shape=jax.ShapeDtypeStruct((V // 2, 2 * D_MODEL), jnp.float32),
    )


@functools.cache
def _build_gather(B: int, V: int):
    assert B % (NW * CHUNK) == 0
    b_per_w = B // NW
    n_chunks = b_per_w // CHUNK
    assert n_chunks % NBUF == 0 and n_chunks >= 2 * NBUF
    mesh = plsc.VectorSubcoreMesh(
        core_axis_name="c", subcore_axis_name="s",
        num_cores=NUM_CORES, num_subcores=NUM_SUBCORES)

    @functools.partial(
        pl.kernel,
        out_type=jax.ShapeDtypeStruct((B, ROW_PAD), jnp.float32),
        mesh=mesh,
        scratch_types=[
            pltpu.VMEM((b_per_w,), jnp.int32),
            [pltpu.VMEM((CHUNK, D_MODEL), jnp.float32)] * NBUF,
            [pltpu.SemaphoreType.DMA] * NBUF,
            [pltpu.SemaphoreType.DMA] * NBUF,
        ],
        compiler_params=pltpu.CompilerParams(use_tc_tiling_on_sc=False),
    )
    def emb(x_hbm, w_hbm, out_hbm, idx_v, rows, gsem, ssem):
        wid = lax.axis_index("s") * NUM_CORES + lax.axis_index("c")
        base = wid * b_per_w

        def gather(g, b):
            pltpu.async_copy(
                w_hbm.at[idx_v.at[pl.ds(g * CHUNK, CHUNK)]], rows[b], gsem[b])

        def wait_gather(b):
            pltpu.make_async_copy(
                w_hbm.at[idx_v.at[pl.ds(0, CHUNK)]], rows[b], gsem[b]).wait()

        def scatter(g, b):
            pltpu.async_copy(
                rows[b],
                out_hbm.at[pl.ds(base + g * CHUNK, CHUNK), pl.ds(0, D_MODEL)],
                ssem[b])

        def wait_scatter(b):
            pltpu.make_async_copy(
                rows[b],
                out_hbm.at[pl.ds(base, CHUNK), pl.ds(0, D_MODEL)],
                ssem[b]).wait()

        pltpu.sync_copy(x_hbm.at[pl.ds(base, b_per_w)], idx_v)

        for b in range(2):
            gather(b, b)

        @pl.loop(0, n_chunks, step=NBUF)
        def _chunks(g0):
            for b in range(NBUF):
                g = g0 + b
                pf = (b + 2) % NBUF
                @pl.when(g + 2 < n_chunks)
                def _():
                    @pl.when(g >= 2)
                    def _():
                        wait_scatter(pf)
                    gather(g + 2, pf)

                wait_gather(b)
                scatter(g, b)

        for b in range(NBUF):
            wait_scatter(b)

    return emb


def kernel(x, weight):
    batch, seq = x.shape
    vocab, _ = weight.shape
    wt = jnp.swapaxes(weight, 0, 1)
    staged = _build_pad_scale(vocab)(wt).reshape(vocab, D_MODEL)
    flat = x.reshape(-1).astype(jnp.int32)
    out = _build_gather(batch * seq, vocab)(flat, staged)
    return out[:, :D_MODEL].reshape(batch, seq, D_MODEL)
